# baseline (speedup 1.0000x reference)
import functools

import jax
import jax.numpy as jnp
from jax import lax
from jax.experimental import pallas as pl
from jax.experimental.pallas import tpu as pltpu
from jax.experimental.pallas import tpu_sc as plsc

N_PLAYER = 1000000
S = 8
B = 16384
T = 20
D = 16

NC = 2
NS = 16
NW = NC * NS
BPW = B // NW
NSUB = 4
SUB = BPW // NSUB
ROWS = SUB * T

_mesh = plsc.VectorSubcoreMesh(core_axis_name="c", subcore_axis_name="s")


@functools.partial(
    pl.kernel,
    out_type=jax.ShapeDtypeStruct((B, D), jnp.float32),
    mesh=_mesh,
    scratch_types=[
        pltpu.VMEM((NSUB, ROWS), jnp.int32),
        pltpu.VMEM((ROWS, D), jnp.float32),
        pltpu.VMEM((BPW, D), jnp.float32),
        pltpu.SemaphoreType.DMA,
    ],
    compiler_params=pltpu.CompilerParams(use_tc_tiling_on_sc=False),
)
def _team_sum_kernel(team_hbm, table_hbm, out_hbm, idx_v, rows_v, out_v, sem):
    wid = lax.axis_index("s") * NC + lax.axis_index("c")
    pltpu.sync_copy(team_hbm.at[wid], idx_v)
    for k in range(NSUB):
        pltpu.async_copy(table_hbm.at[idx_v.at[k]], rows_v, sem).wait()

        def acc(j, _, k=k):
            r = j * T
            a = rows_v[r, :]
            for t in range(1, T):
                a = a + rows_v[r + t, :]
            out_v[k * SUB + j, :] = a
            return 0

        lax.fori_loop(0, SUB, acc, 0)
    pltpu.sync_copy(out_v, out_hbm.at[pl.ds(wid * BPW, BPW)])


def kernel(team, num_samples, mu, rho):
    sigma = jnp.log1p(jnp.exp(rho))
    eps = jax.random.normal(jax.random.key(42), (S, N_PLAYER), dtype=jnp.float32)
    scaled = mu[None, :] + eps * sigma[None, :]
    table = jnp.pad(scaled, ((0, D - S), (0, 0))).T
    team_r = team.reshape(NW, NSUB, ROWS)
    out = _team_sum_kernel(team_r, table)
    one = (jnp.asarray(num_samples) - (S - 1)).astype(jnp.float32)
    return out[:, :S].T * one


# speedup vs baseline: 1.0116x; 1.0116x over previous
import functools

import jax
import jax.numpy as jnp
import numpy as np
from jax import lax
from jax.experimental import pallas as pl
from jax.experimental.pallas import tpu as pltpu
from jax.experimental.pallas import tpu_sc as plsc
from jax.extend.random import threefry2x32_p

N_PLAYER = 1000000
S = 8
B = 16384
T = 20
D = 16

NC = 2
NS = 16
NW = NC * NS
BPW = B // NW
NSUB = 4
SUB = BPW // NSUB
ROWS = SUB * T

_mesh = plsc.VectorSubcoreMesh(core_axis_name="c", subcore_axis_name="s")


@functools.partial(
    pl.kernel,
    out_type=jax.ShapeDtypeStruct((B, D), jnp.float32),
    mesh=_mesh,
    scratch_types=[
        pltpu.VMEM((NSUB, ROWS), jnp.int32),
        pltpu.VMEM((ROWS, D), jnp.float32),
        pltpu.VMEM((BPW, D), jnp.float32),
        pltpu.SemaphoreType.DMA,
    ],
    compiler_params=pltpu.CompilerParams(use_tc_tiling_on_sc=False),
)
def _team_sum_kernel(team_hbm, table_hbm, out_hbm, idx_v, rows_v, out_v, sem):
    wid = lax.axis_index("s") * NC + lax.axis_index("c")
    pltpu.sync_copy(team_hbm.at[wid], idx_v)
    for k in range(NSUB):
        pltpu.async_copy(table_hbm.at[idx_v.at[k]], rows_v, sem).wait()

        def acc(j, _, k=k):
            r = j * T
            a = rows_v[r, :]
            for t in range(1, T):
                a = a + rows_v[r + t, :]
            out_v[k * SUB + j, :] = a
            return 0

        lax.fori_loop(0, SUB, acc, 0)
    pltpu.sync_copy(out_v, out_hbm.at[pl.ds(wid * BPW, BPW)])


def _eps_transposed():
    kd = jax.random.key_data(jax.random.key(42))
    k1 = kd[0].astype(jnp.uint32)
    k2 = kd[1].astype(jnp.uint32)
    p = jnp.arange(N_PLAYER, dtype=jnp.uint32)[:, None]
    s = jnp.arange(S, dtype=jnp.uint32)[None, :]
    c2 = s * np.uint32(N_PLAYER) + p
    b1, b2 = threefry2x32_p.bind(k1, k2, jnp.zeros_like(c2), c2)
    bits = b1 ^ b2
    fb = lax.shift_right_logical(bits, jnp.uint32(9))
    fb = lax.bitwise_or(fb, jnp.uint32(np.array(1.0, np.float32).view(np.uint32)))
    floats = lax.bitcast_convert_type(fb, jnp.float32) - 1.0
    lo = np.nextafter(np.float32(-1.0), np.float32(0.0), dtype=np.float32)
    u = lax.max(jnp.float32(lo), floats * (np.float32(1.0) - lo) + lo)
    return np.float32(np.sqrt(2)) * lax.erf_inv(u)


def kernel(team, num_samples, mu, rho):
    sigma = jnp.log1p(jnp.exp(rho))
    scaled = mu[:, None] + _eps_transposed() * sigma[:, None]
    table = jnp.concatenate(
        [scaled, jnp.zeros((N_PLAYER, D - S), jnp.float32)], axis=1)
    team_r = team.reshape(NW, NSUB, ROWS)
    out = _team_sum_kernel(team_r, table)
    one = (jnp.asarray(num_samples) - (S - 1)).astype(jnp.float32)
    return out[:, :S].T * one


# speedup vs baseline: 1.0833x; 1.0709x over previous
import functools

import jax
import jax.numpy as jnp
import numpy as np
from jax import lax
from jax.experimental import pallas as pl
from jax.experimental.pallas import tpu as pltpu
from jax.experimental.pallas import tpu_sc as plsc
from jax.extend.random import threefry2x32_p

N_PLAYER = 1000000
S = 8
B = 16384
T = 20
D = 16

NC = 2
NS = 16
NW = NC * NS
BPW = B // NW
NSUB = 4
SUB = BPW // NSUB
ROWS = SUB * T

_mesh = plsc.VectorSubcoreMesh(core_axis_name="c", subcore_axis_name="s")


@functools.partial(
    pl.kernel,
    out_type=jax.ShapeDtypeStruct((B, D), jnp.float32),
    mesh=_mesh,
    scratch_types=[
        pltpu.VMEM((NSUB, ROWS), jnp.int32),
        pltpu.VMEM((ROWS, D), jnp.float32),
        pltpu.VMEM((BPW, D), jnp.float32),
        pltpu.SemaphoreType.DMA,
    ],
    compiler_params=pltpu.CompilerParams(use_tc_tiling_on_sc=False),
)
def _team_sum_kernel(team_hbm, table_hbm, out_hbm, idx_v, rows_v, out_v, sem):
    wid = lax.axis_index("s") * NC + lax.axis_index("c")
    pltpu.sync_copy(team_hbm.at[wid], idx_v)
    for k in range(NSUB):
        pltpu.async_copy(table_hbm.at[idx_v.at[k]], rows_v, sem).wait()

        def acc(j, _, k=k):
            r = j * T
            a = rows_v[r, :]
            for t in range(1, T):
                a = a + rows_v[r + t, :]
            out_v[k * SUB + j, :] = a
            return 0

        lax.fori_loop(0, SUB, acc, 0)
    pltpu.sync_copy(out_v, out_hbm.at[pl.ds(wid * BPW, BPW)])


def _eps_transposed_flat():
    kd = jax.random.key_data(jax.random.key(42))
    k1 = kd[0].astype(jnp.uint32)
    k2 = kd[1].astype(jnp.uint32)
    j = lax.iota(jnp.uint32, N_PLAYER * D)
    p = lax.shift_right_logical(j, jnp.uint32(4))
    s = lax.bitwise_and(j, jnp.uint32(D - 1))
    c2 = s * np.uint32(N_PLAYER) + p
    b1, b2 = threefry2x32_p.bind(k1, k2, jnp.zeros_like(c2), c2)
    bits = b1 ^ b2
    fb = lax.shift_right_logical(bits, jnp.uint32(9))
    fb = lax.bitwise_or(fb, jnp.uint32(np.array(1.0, np.float32).view(np.uint32)))
    floats = lax.bitcast_convert_type(fb, jnp.float32) - 1.0
    lo = np.nextafter(np.float32(-1.0), np.float32(0.0), dtype=np.float32)
    u = lax.max(jnp.float32(lo), floats * (np.float32(1.0) - lo) + lo)
    return np.float32(np.sqrt(2)) * lax.erf_inv(u)


def _rep(x):
    return jnp.broadcast_to(x[:, None], (N_PLAYER, D)).reshape(N_PLAYER * D)


def kernel(team, num_samples, mu, rho):
    sigma = jnp.log1p(jnp.exp(rho))
    table = (_rep(mu) + _eps_transposed_flat() * _rep(sigma)).reshape(N_PLAYER, D)
    team_r = team.reshape(NW, NSUB, ROWS)
    out = _team_sum_kernel(team_r, table)
    one = (jnp.asarray(num_samples) - (S - 1)).astype(jnp.float32)
    return out[:, :S].T * one


# speedup vs baseline: 28.2819x; 26.1070x over previous
import functools

import jax
import jax.numpy as jnp
import numpy as np
from jax import lax
from jax.experimental import pallas as pl
from jax.experimental.pallas import tpu as pltpu
from jax.experimental.pallas import tpu_sc as plsc

N_PLAYER = 1000000
S = 8
B = 16384
T = 20

NC = 2
NS = 16
NW = NC * NS
TPW = (B * T) // NW

_K1 = np.uint32(0)
_K2 = np.uint32(42)

_mesh = plsc.VectorSubcoreMesh(core_axis_name="c", subcore_axis_name="s")


@functools.partial(
    pl.kernel,
    out_type=(
        jax.ShapeDtypeStruct((B * T,), jnp.float32),
        jax.ShapeDtypeStruct((B * T,), jnp.float32),
    ),
    mesh=_mesh,
    scratch_types=[
        pltpu.VMEM((TPW,), jnp.int32),
        pltpu.VMEM((TPW,), jnp.float32),
        pltpu.VMEM((TPW,), jnp.float32),
        pltpu.SemaphoreType.DMA,
        pltpu.SemaphoreType.DMA,
    ],
    compiler_params=pltpu.CompilerParams(use_tc_tiling_on_sc=False),
)
def _gather_sc(teamt_hbm, mu_hbm, rho_hbm, mug_hbm, rhog_hbm,
               idx_v, a_v, b_v, sem_a, sem_b):
    wid = lax.axis_index("s") * NC + lax.axis_index("c")
    base = wid * TPW
    pltpu.sync_copy(teamt_hbm.at[pl.ds(base, TPW)], idx_v)
    ca = pltpu.async_copy(mu_hbm.at[idx_v], a_v, sem_a)
    cb = pltpu.async_copy(rho_hbm.at[idx_v], b_v, sem_b)
    ca.wait()
    cb.wait()
    pltpu.sync_copy(a_v, mug_hbm.at[pl.ds(base, TPW)])
    pltpu.sync_copy(b_v, rhog_hbm.at[pl.ds(base, TPW)])


def _tf_bits(c2):
    ks0, ks1 = _K1, _K2
    ks2 = np.uint32(ks0 ^ ks1 ^ np.uint32(0x1BD11BDA))
    x0 = jnp.full_like(c2, ks0)
    x1 = c2 + ks1
    r_a = (13, 15, 26, 6)
    r_b = (17, 29, 16, 24)
    sched = [(ks1, ks2, 1), (ks2, ks0, 2), (ks0, ks1, 3),
             (ks1, ks2, 4), (ks2, ks0, 5)]
    rots = [r_a, r_b, r_a, r_b, r_a]
    for (a, b, inc), rs in zip(sched, rots):
        for r in rs:
            x0 = x0 + x1
            x1 = (x1 << np.uint32(r)) | (x1 >> np.uint32(32 - r))
            x1 = x0 ^ x1
        x0 = x0 + a
        x1 = x1 + np.uint32(b + np.uint32(inc))
    return x0 ^ x1


_ERF_SMALL = [np.float32(v) for v in (
    2.81022636e-08, 3.43273939e-07, -3.5233877e-06,
    -4.39150654e-06, 0.00021858087, -0.00125372503,
    -0.00417768164, 0.246640727, 1.50140941)]
_ERF_LARGE = [np.float32(v) for v in (
    -0.000200214257, 0.000100950558, 0.00134934322,
    -0.00367342844, 0.00573950773, -0.0076224613,
    0.00943887047, 1.00167406, 2.83297682)]


def _erfinv(x):
    w = -jnp.log1p(-x * x)
    small = w < np.float32(5.0)
    ws = w - np.float32(2.5)
    wl = jnp.sqrt(w) - np.float32(3.0)
    ps = _ERF_SMALL[0]
    for c in _ERF_SMALL[1:]:
        ps = ps * ws + c
    pL = _ERF_LARGE[0]
    for c in _ERF_LARGE[1:]:
        pL = pL * wl + c
    return jnp.where(small, ps, pL) * x


_U_LO = np.nextafter(np.float32(-1.0), np.float32(0.0), dtype=np.float32)
_U_SCALE = np.float32(np.float32(1.0) - _U_LO)
_SQRT2 = np.float32(np.sqrt(2.0))


def _eps_at(c2):
    bits = _tf_bits(c2)
    fb = (bits >> np.uint32(9)) | np.uint32(0x3F800000)
    f = lax.bitcast_convert_type(fb, jnp.float32) - np.float32(1.0)
    u = lax.max(jnp.asarray(_U_LO), f * _U_SCALE + _U_LO)
    return _SQRT2 * _erfinv(u)


TCB = 256


def _perf_tc_body(teamt_ref, mug_ref, rhog_ref, out_ref):
    p = teamt_ref[...].astype(jnp.uint32)
    mug = mug_ref[...]
    sig = jnp.log1p(jnp.exp(rhog_ref[...]))
    for s in range(S):
        eps = _eps_at(p + np.uint32(s * N_PLAYER))
        out_ref[s, :] = jnp.sum(mug + eps * sig, axis=0)


def kernel(team, num_samples, mu, rho):
    teamt = team.T.reshape(B * T)
    mug, rhog = _gather_sc(teamt, mu, rho)
    perf = pl.pallas_call(
        _perf_tc_body,
        grid=(B // TCB,),
        in_specs=[
            pl.BlockSpec((T, TCB), lambda i: (0, i)),
            pl.BlockSpec((T, TCB), lambda i: (0, i)),
            pl.BlockSpec((T, TCB), lambda i: (0, i)),
        ],
        out_specs=pl.BlockSpec((S, TCB), lambda i: (0, i)),
        out_shape=jax.ShapeDtypeStruct((S, B), jnp.float32),
    )(teamt.reshape(T, B), mug.reshape(T, B), rhog.reshape(T, B))
    one = (jnp.asarray(num_samples) - (S - 1)).astype(jnp.float32)
    return perf * one
